# Initial kernel scaffold; baseline (speedup 1.0000x reference)
import jax
import jax.numpy as jnp
from jax.experimental import pallas as pl


def kernel(node_embs, mask, edge_index, edge_type, scorer_self, gates_self_W, gates_self_U, gates_self_b, scorer_rel, gates_rel_W, gates_rel_U, gates_rel_b, W_init_self, W_init_rel):
    raise NotImplementedError("write your pallas kernel here")


# speedup vs baseline: 18.3427x; 18.3427x over previous
import functools

import jax
import jax.numpy as jnp
from jax import lax
from jax.experimental import pallas as pl
from jax.experimental.pallas import tpu as pltpu
from jax.experimental.pallas import tpu_sc as plsc

N = 10000
E = 320000
D = 128
R = 4
K = 128
NH = R + 1
HP = 8

NC = 2
NS = 16
EP = E // (NC * NS)
HROWS = 79
DEG_BINS = R * N
DEG_PAD = 40960
CE = 2000
CH = 128
N2 = 10048
NL = 2 * R
HALF = 5120
TPT = HALF // NS
ACAP = EP + NL * CH + 48
ZROW = N

@functools.cache
def _sc_mesh():
    return plsc.VectorSubcoreMesh(core_axis_name="c", subcore_axis_name="s",
                                  num_cores=NC, num_subcores=NS)


def _deg_body(row_hbm, typ_hbm, deg_hbm, rbuf, tbuf, hlist, ones, zbuf, degacc):
    c = lax.axis_index("c")
    s = lax.axis_index("s")
    e0 = c * (E // NC) + s * EP

    for i in range(CH // 16):
        ones[pl.ds(i * 16, 16)] = jnp.ones((16,), jnp.float32)

    def zb(i, _):
        zbuf[pl.ds(i * 16, 16)] = jnp.zeros((16,), jnp.float32)
        return 0
    lax.fori_loop(0, (DEG_PAD // NS) // 16, zb, 0)

    pltpu.sync_copy(row_hbm.at[pl.ds(e0, EP)], rbuf)
    pltpu.sync_copy(typ_hbm.at[pl.ds(e0, EP)], tbuf)

    def hrow(j, _):
        for u in range(CH // 16):
            f = j * CH + u * 16
            hlist[j, pl.ds(u * 16, 16)] = (
                tbuf[pl.ds(f, 16)] * N + rbuf[pl.ds(f, 16)])
        return 0
    lax.fori_loop(0, HROWS - 1, hrow, 0)
    hlist[HROWS - 1, pl.ds(0, 16)] = (
        tbuf[pl.ds(EP - 16, 16)] * N + rbuf[pl.ds(EP - 16, 16)])
    for u in range(1, CH // 16):
        hlist[HROWS - 1, pl.ds(u * 16, 16)] = jnp.full((16,), DEG_BINS,
                                                       jnp.int32)

    pltpu.sync_copy(zbuf, degacc.at[pl.ds(s * (DEG_PAD // NS), DEG_PAD // NS)])
    plsc.subcore_barrier()

    def scat(j, _):
        pltpu.sync_copy(ones, degacc.at[hlist.at[j]], add=True)
        return 0
    lax.fori_loop(0, HROWS, scat, 0)
    plsc.subcore_barrier()

    sl = pl.ds(s * (DEG_PAD // NS), DEG_PAD // NS)
    pltpu.sync_copy(degacc.at[sl], deg_hbm.at[c, sl])


@functools.cache
def _deg_kernel():
    return pl.kernel(
        _deg_body,
        out_type=jax.ShapeDtypeStruct((NC, DEG_PAD), jnp.float32),
        mesh=_sc_mesh(),
        compiler_params=pltpu.CompilerParams(needs_layout_passes=False),
        scratch_types=[
            pltpu.VMEM((EP,), jnp.int32),
            pltpu.VMEM((EP,), jnp.int32),
            pltpu.VMEM((HROWS, CH), jnp.int32),
            pltpu.VMEM((CH,), jnp.float32),
            pltpu.VMEM((DEG_PAD // NS,), jnp.float32),
            pltpu.VMEM_SHARED((DEG_PAD,), jnp.float32),
        ],
    )


def _deg_call(row, typ):
    return _deg_kernel()(row, typ)


def _topk_body(x_ref, p_ref, mask_ref, w_ref, idx_ref, s_ref):
    p = p_ref[...]
    nrm = jnp.sqrt(jnp.sum(p * p, axis=0, keepdims=True))
    sc = jnp.dot(x_ref[...], p, preferred_element_type=jnp.float32)
    s_ref[...] = sc / (nrm + 1e-8) + mask_ref[...]
    iota = lax.broadcasted_iota(jnp.int32, (N, HP), 0)

    def body(i, _):
        sv = s_ref[...]
        m = jnp.max(sv, axis=0, keepdims=True)
        am = jnp.min(jnp.where(sv == m, iota, N), axis=0, keepdims=True)
        w_ref[pl.ds(i, 1), :] = jnp.tanh(m)
        idx_ref[pl.ds(i, 1), :] = am
        s_ref[...] = jnp.where(iota == am, -jnp.inf, sv)
        return 0
    lax.fori_loop(0, K, body, 0)


def _topk_call(x, p, mask2d):
    return pl.pallas_call(
        _topk_body,
        out_shape=[jax.ShapeDtypeStruct((K, HP), jnp.float32),
                   jax.ShapeDtypeStruct((K, HP), jnp.int32)],
        scratch_shapes=[pltpu.VMEM((N, HP), jnp.float32)],
    )(x, p, mask2d)


def _gru_body(idx_ref, w_ref, x_ref, W_ref, U_ref, b_ref, q_ref, out_ref,
              sel_ref):
    h = pl.program_id(0)

    def gather(i, _):
        r = idx_ref[i, h]
        sel_ref[pl.ds(i, 1), :] = x_ref[pl.ds(r, 1), :] * w_ref[i, h]
        return 0
    lax.fori_loop(0, K, gather, 0)

    sw = sel_ref[...]
    q = q_ref[0]

    def nt(a, b):
        return lax.dot_general(a, b, (((1,), (1,)), ((), ())),
                               preferred_element_type=jnp.float32)

    def nn(a, b):
        return jnp.dot(a, b, preferred_element_type=jnp.float32)

    upd = jax.nn.sigmoid(nt(W_ref[0, 0], sw) + nn(U_ref[0, 0], q)
                         + b_ref[0, 0])
    rst = jax.nn.sigmoid(nt(W_ref[0, 1], sw) + nn(U_ref[0, 1], q)
                         + b_ref[0, 1])
    hc = jnp.tanh(nt(W_ref[0, 2], sw) + nn(U_ref[0, 2], rst * q)
                  + b_ref[0, 2])
    out_ref[0] = (1.0 - upd) * q + upd * hc


def _gru_call(idx, wv, x, Wall, Uall, ball, Qinit):
    return pl.pallas_call(
        _gru_body,
        grid=(NH,),
        in_specs=[
            pl.BlockSpec(memory_space=pltpu.SMEM),
            pl.BlockSpec(memory_space=pltpu.SMEM),
            pl.BlockSpec((N, D), lambda h: (0, 0)),
            pl.BlockSpec((1, 3, D, D), lambda h: (h, 0, 0, 0)),
            pl.BlockSpec((1, 3, D, D), lambda h: (h, 0, 0, 0)),
            pl.BlockSpec((1, 3, D, D), lambda h: (h, 0, 0, 0)),
            pl.BlockSpec((1, D, D), lambda h: (h, 0, 0)),
        ],
        out_specs=pl.BlockSpec((1, D, D), lambda h: (h, 0, 0)),
        out_shape=jax.ShapeDtypeStruct((NH, D, D), jnp.float32),
        scratch_shapes=[pltpu.VMEM((K, D), jnp.float32)],
    )(idx, wv, x, Wall, Uall, ball, Qinit)


BN = 2512


def _mm_body(x_ref, q_ref, degp_ref, y_ref):
    r = pl.program_id(0)
    b = pl.program_id(1)
    xq = jnp.dot(x_ref[...], q_ref[0], preferred_element_type=jnp.float32)
    d2 = degp_ref[0, 0] + degp_ref[1, 0]
    dis = jnp.where(d2 > 0, lax.rsqrt(d2), 0.0)
    scale = jnp.where(r == 0, jnp.ones_like(dis), dis)
    rid = b * BN + lax.broadcasted_iota(jnp.int32, (BN, 1), 0)
    y_ref[0] = jnp.where(rid < N, xq * scale, 0.0)


def _mm_call(x, Qall, degp):
    return pl.pallas_call(
        _mm_body,
        grid=(NH, N2 // BN),
        in_specs=[
            pl.BlockSpec((BN, D), lambda r, b: (b, 0)),
            pl.BlockSpec((1, D, D), lambda r, b: (r, 0, 0)),
            pl.BlockSpec((2, 1, BN, 1),
                         lambda r, b: (0, jnp.maximum(r - 1, 0), b, 0)),
        ],
        out_specs=pl.BlockSpec((1, BN, D), lambda r, b: (r, b, 0)),
        out_shape=jax.ShapeDtypeStruct((NH, N2, D), jnp.float32),
    )(x, Qall, degp)


def _edge_body(row_hbm, col_hbm, typ_hbm, ytc_hbm, parts_hbm,
               rbuf, cbuf, tbuf, garena, sarena, gbuf, sbuf, dbuf, acc, sem):
    c = lax.axis_index("c")
    s = lax.axis_index("s")
    e0 = c * (E // NC) + s * EP

    cnts = [jnp.int32(0)] * NL
    for ch in range(EP // CE):
        pltpu.sync_copy(row_hbm.at[pl.ds(e0 + ch * CE, CE)], rbuf)
        pltpu.sync_copy(typ_hbm.at[pl.ds(e0 + ch * CE, CE)], tbuf)

        def count_body(i, cn):
            rv = rbuf[pl.ds(i * 16, 16)]
            tv = tbuf[pl.ds(i * 16, 16)]
            hi = rv >= HALF
            new = []
            for t in range(R):
                for h in range(2):
                    m = (tv == t) & (hi if h else jnp.logical_not(hi))
                    new.append(cn[t * 2 + h] + jnp.max(
                        plsc.all_reduce_population_count(m)))
            return tuple(new)
        cnts = list(lax.fori_loop(0, CE // 16, count_body, tuple(cnts)))

    off = [jnp.int32(0)]
    for k in range(NL):
        off.append(off[k] + ((cnts[k] + CH - 1) & ~jnp.int32(CH - 1)))

    def pre(i, _):
        garena[pl.ds(i * 16, 16)] = jnp.full((16,), ZROW, jnp.int32)
        sarena[pl.ds(i * 16, 16)] = jnp.zeros((16,), jnp.int32)
        return 0
    lax.fori_loop(0, ACAP // 16, pre, 0)

    curs = list(off[:NL])
    for ch in range(EP // CE):
        pltpu.sync_copy(row_hbm.at[pl.ds(e0 + ch * CE, CE)], rbuf)
        pltpu.sync_copy(col_hbm.at[pl.ds(e0 + ch * CE, CE)], cbuf)
        pltpu.sync_copy(typ_hbm.at[pl.ds(e0 + ch * CE, CE)], tbuf)

        def fill_body(i, cu):
            rv = rbuf[pl.ds(i * 16, 16)]
            cv = cbuf[pl.ds(i * 16, 16)]
            tv = tbuf[pl.ds(i * 16, 16)]
            hi = rv >= HALF
            new = []
            for t in range(R):
                for h in range(2):
                    k = t * 2 + h
                    m = (tv == t) & (hi if h else jnp.logical_not(hi))
                    plsc.store_compressed(garena.at[pl.ds(cu[k], 16)],
                                          cv + (t + 1) * N2, mask=m)
                    plsc.store_compressed(sarena.at[pl.ds(cu[k], 16)],
                                          rv - h * HALF, mask=m)
                    new.append(cu[k] + jnp.max(
                        plsc.all_reduce_population_count(m)))
            return tuple(new)
        curs = list(lax.fori_loop(0, CE // 16, fill_body, tuple(curs)))

    base = s * TPT
    for h in range(2):
        for t in range(R):
            k = t * 2 + h

            def zdb(j, _):
                for u in range(D // 16):
                    dbuf[j, pl.ds(u * 16, 16)] = jnp.zeros((16,), jnp.float32)
                return 0
            lax.fori_loop(0, CH, zdb, 0)
            for z in range(TPT // CH):
                pltpu.sync_copy(dbuf, acc.at[pl.ds(base + z * CH, CH)])
            if TPT % CH:
                pltpu.sync_copy(dbuf.at[pl.ds(0, TPT % CH)],
                                acc.at[pl.ds(base + (TPT // CH) * CH,
                                             TPT % CH)])
            plsc.subcore_barrier()

            nch = (off[k + 1] - off[k]) // CH

            def chunk(j, _):
                o = off[k] + j * CH
                for u in range(CH // 16):
                    gbuf[pl.ds(u * 16, 16)] = garena[pl.ds(o + u * 16, 16)]
                    sbuf[pl.ds(u * 16, 16)] = sarena[pl.ds(o + u * 16, 16)]
                pltpu.async_copy(ytc_hbm.at[gbuf], dbuf, sem).wait()
                pltpu.sync_copy(dbuf, acc.at[sbuf], add=True)
                return 0
            lax.fori_loop(0, nch, chunk, 0)
            plsc.subcore_barrier()

            pltpu.sync_copy(acc.at[pl.ds(base, TPT)],
                            parts_hbm.at[c, t, pl.ds(h * HALF + base, TPT)])
            plsc.subcore_barrier()


@functools.cache
def _edge_kernel():
    return pl.kernel(
        _edge_body,
        out_type=jax.ShapeDtypeStruct((NC, R, 2 * HALF, D), jnp.float32),
        mesh=_sc_mesh(),
        compiler_params=pltpu.CompilerParams(needs_layout_passes=False),
        scratch_types=[
            pltpu.VMEM((CE,), jnp.int32),
            pltpu.VMEM((CE,), jnp.int32),
            pltpu.VMEM((CE,), jnp.int32),
            pltpu.VMEM((ACAP,), jnp.int32),
            pltpu.VMEM((ACAP,), jnp.int32),
            pltpu.VMEM((CH,), jnp.int32),
            pltpu.VMEM((CH,), jnp.int32),
            pltpu.VMEM((CH, D), jnp.float32),
            pltpu.VMEM_SHARED((HALF, D), jnp.float32),
            pltpu.SemaphoreType.DMA,
        ],
    )


def _edge_call(row, col, typ, ytc):
    return _edge_kernel()(row, col, typ, ytc)


BN4 = 1000


def _comb_body(y_ref, parts_ref, degp_ref, o_ref):
    accv = y_ref[0]
    d2 = degp_ref[0] + degp_ref[1]
    dis = jnp.where(d2 > 0, lax.rsqrt(d2), 0.0)
    for t in range(R):
        accv = accv + (parts_ref[0, t] + parts_ref[1, t]) * dis[t]
    o_ref[...] = jnp.maximum(accv, 0.0)


def _comb_call(y, parts, degp):
    return pl.pallas_call(
        _comb_body,
        grid=(N // BN4,),
        in_specs=[
            pl.BlockSpec((1, BN4, D), lambda b: (0, b, 0)),
            pl.BlockSpec((NC, R, BN4, D), lambda b: (0, 0, b, 0)),
            pl.BlockSpec((NC, R, BN4, 1), lambda b: (0, 0, b, 0)),
        ],
        out_specs=pl.BlockSpec((BN4, D), lambda b: (b, 0)),
        out_shape=jax.ShapeDtypeStruct((N, D), jnp.float32),
    )(y, parts, degp)


def kernel(node_embs, mask, edge_index, edge_type, scorer_self, gates_self_W,
           gates_self_U, gates_self_b, scorer_rel, gates_rel_W, gates_rel_U,
           gates_rel_b, W_init_self, W_init_rel):
    x = node_embs
    row = edge_index[0].astype(jnp.int32)
    col = edge_index[1].astype(jnp.int32)
    typ = edge_type.astype(jnp.int32)

    p = jnp.concatenate(
        [scorer_self, jnp.moveaxis(scorer_rel, 0, 2).reshape(D, R)], axis=1)
    p = jnp.pad(p, ((0, 0), (0, HP - NH)))
    Wall = jnp.concatenate([gates_self_W[None], gates_rel_W], axis=0)
    Uall = jnp.concatenate([gates_self_U[None], gates_rel_U], axis=0)
    ball = jnp.concatenate([gates_self_b[None], gates_rel_b], axis=0)
    Qinit = jnp.concatenate([W_init_self[None], W_init_rel], axis=0)

    deg_raw = _deg_call(row, typ)
    wv, idx = _topk_call(x, p, mask.reshape(N, 1))
    Qall = _gru_call(idx, wv, x, Wall, Uall, ball, Qinit)
    degp = deg_raw[:, :DEG_BINS].reshape(NC, R, N, 1)
    y = _mm_call(x, Qall, degp)
    parts = _edge_call(row, col, typ, y.reshape(NH * N2, D))
    return _comb_call(y, parts, degp)


# speedup vs baseline: 18.9116x; 1.0310x over previous
import functools

import jax
import jax.numpy as jnp
from jax import lax
from jax.experimental import pallas as pl
from jax.experimental.pallas import tpu as pltpu
from jax.experimental.pallas import tpu_sc as plsc

N = 10000
E = 320000
D = 128
R = 4
K = 128
NH = R + 1
HP = 8

NC = 2
NS = 16
EP = E // (NC * NS)
HROWS = 79
DEG_BINS = R * N
DEG_PAD = 40960
CE = 2000
CH = 128
N2 = 10048
NL = 2 * R
HALF = 5120
TPT = HALF // NS
ACAP = EP + NL * CH + 48
ZROW = N

@functools.cache
def _sc_mesh():
    return plsc.VectorSubcoreMesh(core_axis_name="c", subcore_axis_name="s",
                                  num_cores=NC, num_subcores=NS)


def _deg_body(row_hbm, typ_hbm, deg_hbm, rbuf, tbuf, hlist, ones, zbuf, degacc):
    c = lax.axis_index("c")
    s = lax.axis_index("s")
    e0 = c * (E // NC) + s * EP

    for i in range(CH // 16):
        ones[pl.ds(i * 16, 16)] = jnp.ones((16,), jnp.float32)

    def zb(i, _):
        zbuf[pl.ds(i * 16, 16)] = jnp.zeros((16,), jnp.float32)
        return 0
    lax.fori_loop(0, (DEG_PAD // NS) // 16, zb, 0)

    pltpu.sync_copy(row_hbm.at[pl.ds(e0, EP)], rbuf)
    pltpu.sync_copy(typ_hbm.at[pl.ds(e0, EP)], tbuf)

    def hrow(j, _):
        for u in range(CH // 16):
            f = j * CH + u * 16
            hlist[j, pl.ds(u * 16, 16)] = (
                tbuf[pl.ds(f, 16)] * N + rbuf[pl.ds(f, 16)])
        return 0
    lax.fori_loop(0, HROWS - 1, hrow, 0)
    hlist[HROWS - 1, pl.ds(0, 16)] = (
        tbuf[pl.ds(EP - 16, 16)] * N + rbuf[pl.ds(EP - 16, 16)])
    for u in range(1, CH // 16):
        hlist[HROWS - 1, pl.ds(u * 16, 16)] = jnp.full((16,), DEG_BINS,
                                                       jnp.int32)

    pltpu.sync_copy(zbuf, degacc.at[pl.ds(s * (DEG_PAD // NS), DEG_PAD // NS)])
    plsc.subcore_barrier()

    def scat(j, _):
        pltpu.sync_copy(ones, degacc.at[hlist.at[j]], add=True)
        return 0
    lax.fori_loop(0, HROWS, scat, 0)
    plsc.subcore_barrier()

    sl = pl.ds(s * (DEG_PAD // NS), DEG_PAD // NS)
    pltpu.sync_copy(degacc.at[sl], deg_hbm.at[c, sl])


@functools.cache
def _deg_kernel():
    return pl.kernel(
        _deg_body,
        out_type=jax.ShapeDtypeStruct((NC, DEG_PAD), jnp.float32),
        mesh=_sc_mesh(),
        compiler_params=pltpu.CompilerParams(needs_layout_passes=False),
        scratch_types=[
            pltpu.VMEM((EP,), jnp.int32),
            pltpu.VMEM((EP,), jnp.int32),
            pltpu.VMEM((HROWS, CH), jnp.int32),
            pltpu.VMEM((CH,), jnp.float32),
            pltpu.VMEM((DEG_PAD // NS,), jnp.float32),
            pltpu.VMEM_SHARED((DEG_PAD,), jnp.float32),
        ],
    )


def _deg_call(row, typ):
    return _deg_kernel()(row, typ)


def _topk_body(x_ref, p_ref, mask_ref, w_ref, idx_ref, s_ref):
    p = p_ref[...]
    nrm = jnp.sqrt(jnp.sum(p * p, axis=0, keepdims=True))
    sc = jnp.dot(x_ref[...], p, preferred_element_type=jnp.float32)
    s_ref[...] = sc / (nrm + 1e-8) + mask_ref[...]
    iota = lax.broadcasted_iota(jnp.int32, (N, HP), 0)

    def body(i, _):
        sv = s_ref[...]
        m = jnp.max(sv, axis=0, keepdims=True)
        am = jnp.min(jnp.where(sv == m, iota, N), axis=0, keepdims=True)
        w_ref[pl.ds(i, 1), :] = jnp.tanh(m)
        idx_ref[pl.ds(i, 1), :] = am
        s_ref[...] = jnp.where(iota == am, -jnp.inf, sv)
        return 0
    lax.fori_loop(0, K, body, 0)


def _topk_call(x, p, mask2d):
    return pl.pallas_call(
        _topk_body,
        out_shape=[jax.ShapeDtypeStruct((K, HP), jnp.float32),
                   jax.ShapeDtypeStruct((K, HP), jnp.int32)],
        scratch_shapes=[pltpu.VMEM((N, HP), jnp.float32)],
    )(x, p, mask2d)


def _gru_body(idx_ref, w_ref, x_ref, W_ref, U_ref, b_ref, q_ref, out_ref,
              sel_ref):
    h = pl.program_id(0)

    def gather(i, _):
        r = idx_ref[i, h]
        sel_ref[pl.ds(i, 1), :] = x_ref[pl.ds(r, 1), :] * w_ref[i, h]
        return 0
    lax.fori_loop(0, K, gather, 0)

    sw = sel_ref[...]
    q = q_ref[0]

    def nt(a, b):
        return lax.dot_general(a, b, (((1,), (1,)), ((), ())),
                               preferred_element_type=jnp.float32)

    def nn(a, b):
        return jnp.dot(a, b, preferred_element_type=jnp.float32)

    upd = jax.nn.sigmoid(nt(W_ref[0, 0], sw) + nn(U_ref[0, 0], q)
                         + b_ref[0, 0])
    rst = jax.nn.sigmoid(nt(W_ref[0, 1], sw) + nn(U_ref[0, 1], q)
                         + b_ref[0, 1])
    hc = jnp.tanh(nt(W_ref[0, 2], sw) + nn(U_ref[0, 2], rst * q)
                  + b_ref[0, 2])
    out_ref[0] = (1.0 - upd) * q + upd * hc


def _gru_call(idx, wv, x, Wall, Uall, ball, Qinit):
    return pl.pallas_call(
        _gru_body,
        grid=(NH,),
        in_specs=[
            pl.BlockSpec(memory_space=pltpu.SMEM),
            pl.BlockSpec(memory_space=pltpu.SMEM),
            pl.BlockSpec((N, D), lambda h: (0, 0)),
            pl.BlockSpec((1, 3, D, D), lambda h: (h, 0, 0, 0)),
            pl.BlockSpec((1, 3, D, D), lambda h: (h, 0, 0, 0)),
            pl.BlockSpec((1, 3, D, D), lambda h: (h, 0, 0, 0)),
            pl.BlockSpec((1, D, D), lambda h: (h, 0, 0)),
        ],
        out_specs=pl.BlockSpec((1, D, D), lambda h: (h, 0, 0)),
        out_shape=jax.ShapeDtypeStruct((NH, D, D), jnp.float32),
        scratch_shapes=[pltpu.VMEM((K, D), jnp.float32)],
    )(idx, wv, x, Wall, Uall, ball, Qinit)


BN = 2512


def _mm_body(x_ref, q_ref, degp_ref, y_ref):
    r = pl.program_id(0)
    b = pl.program_id(1)
    xq = jnp.dot(x_ref[...], q_ref[0], preferred_element_type=jnp.float32)
    d2 = degp_ref[0, 0] + degp_ref[1, 0]
    dis = jnp.where(d2 > 0, lax.rsqrt(d2), 0.0)
    scale = jnp.where(r == 0, jnp.ones_like(dis), dis)
    rid = b * BN + lax.broadcasted_iota(jnp.int32, (BN, 1), 0)
    y_ref[0] = jnp.where(rid < N, xq * scale, 0.0)


def _mm_call(x, Qall, degp):
    return pl.pallas_call(
        _mm_body,
        grid=(NH, N2 // BN),
        in_specs=[
            pl.BlockSpec((BN, D), lambda r, b: (b, 0)),
            pl.BlockSpec((1, D, D), lambda r, b: (r, 0, 0)),
            pl.BlockSpec((2, 1, BN, 1),
                         lambda r, b: (0, jnp.maximum(r - 1, 0), b, 0)),
        ],
        out_specs=pl.BlockSpec((1, BN, D), lambda r, b: (r, b, 0)),
        out_shape=jax.ShapeDtypeStruct((NH, N2, D), jnp.float32),
    )(x, Qall, degp)


def _edge_body(row_hbm, col_hbm, typ_hbm, ytc_hbm, parts_hbm,
               rbuf, cbuf, tbuf, garena, sarena, gbuf0, sbuf0, dbuf0,
               gbuf1, sbuf1, dbuf1, acc, sem0, sem1):
    c = lax.axis_index("c")
    s = lax.axis_index("s")
    e0 = c * (E // NC) + s * EP

    cnts = [jnp.int32(0)] * NL
    for ch in range(EP // CE):
        pltpu.sync_copy(row_hbm.at[pl.ds(e0 + ch * CE, CE)], rbuf)
        pltpu.sync_copy(typ_hbm.at[pl.ds(e0 + ch * CE, CE)], tbuf)

        def count_body(i, cn):
            rv = rbuf[pl.ds(i * 16, 16)]
            tv = tbuf[pl.ds(i * 16, 16)]
            hi = rv >= HALF
            new = []
            for t in range(R):
                for h in range(2):
                    m = (tv == t) & (hi if h else jnp.logical_not(hi))
                    new.append(cn[t * 2 + h] + jnp.max(
                        plsc.all_reduce_population_count(m)))
            return tuple(new)
        cnts = list(lax.fori_loop(0, CE // 16, count_body, tuple(cnts)))

    off = [jnp.int32(0)]
    for k in range(NL):
        off.append(off[k] + ((cnts[k] + CH - 1) & ~jnp.int32(CH - 1)))

    def pre(i, _):
        garena[pl.ds(i * 16, 16)] = jnp.full((16,), ZROW, jnp.int32)
        sarena[pl.ds(i * 16, 16)] = jnp.zeros((16,), jnp.int32)
        return 0
    lax.fori_loop(0, ACAP // 16, pre, 0)

    curs = list(off[:NL])
    for ch in range(EP // CE):
        pltpu.sync_copy(row_hbm.at[pl.ds(e0 + ch * CE, CE)], rbuf)
        pltpu.sync_copy(col_hbm.at[pl.ds(e0 + ch * CE, CE)], cbuf)
        pltpu.sync_copy(typ_hbm.at[pl.ds(e0 + ch * CE, CE)], tbuf)

        def fill_body(i, cu):
            rv = rbuf[pl.ds(i * 16, 16)]
            cv = cbuf[pl.ds(i * 16, 16)]
            tv = tbuf[pl.ds(i * 16, 16)]
            hi = rv >= HALF
            new = []
            for t in range(R):
                for h in range(2):
                    k = t * 2 + h
                    m = (tv == t) & (hi if h else jnp.logical_not(hi))
                    plsc.store_compressed(garena.at[pl.ds(cu[k], 16)],
                                          cv + (t + 1) * N2, mask=m)
                    plsc.store_compressed(sarena.at[pl.ds(cu[k], 16)],
                                          rv - h * HALF, mask=m)
                    new.append(cu[k] + jnp.max(
                        plsc.all_reduce_population_count(m)))
            return tuple(new)
        curs = list(lax.fori_loop(0, CE // 16, fill_body, tuple(curs)))

    base = s * TPT
    for h in range(2):
        for t in range(R):
            k = t * 2 + h

            def zdb(j, _):
                for u in range(D // 16):
                    dbuf0[j, pl.ds(u * 16, 16)] = jnp.zeros((16,), jnp.float32)
                return 0
            lax.fori_loop(0, CH, zdb, 0)
            for z in range(TPT // CH):
                pltpu.sync_copy(dbuf0, acc.at[pl.ds(base + z * CH, CH)])
            if TPT % CH:
                pltpu.sync_copy(dbuf0.at[pl.ds(0, TPT % CH)],
                                acc.at[pl.ds(base + (TPT // CH) * CH,
                                             TPT % CH)])
            plsc.subcore_barrier()

            nch = (off[k + 1] - off[k]) // CH

            def fill(o, gb, sb):
                for u in range(CH // 16):
                    gb[pl.ds(u * 16, 16)] = garena[pl.ds(o + u * 16, 16)]
                    sb[pl.ds(u * 16, 16)] = sarena[pl.ds(o + u * 16, 16)]

            @pl.when(nch > 0)
            def _():
                fill(off[k], gbuf0, sbuf0)
                pltpu.async_copy(ytc_hbm.at[gbuf0], dbuf0, sem0)

            def pair(m, _):
                j1 = 2 * m + 1
                pltpu.make_async_copy(ytc_hbm.at[gbuf0], dbuf0, sem0).wait()

                @pl.when(j1 < nch)
                def _():
                    fill(off[k] + j1 * CH, gbuf1, sbuf1)
                    pltpu.async_copy(ytc_hbm.at[gbuf1], dbuf1, sem1)
                pltpu.sync_copy(dbuf0, acc.at[sbuf0], add=True)

                @pl.when(j1 < nch)
                def _():
                    pltpu.make_async_copy(ytc_hbm.at[gbuf1], dbuf1,
                                          sem1).wait()

                    @pl.when(j1 + 1 < nch)
                    def _():
                        fill(off[k] + (j1 + 1) * CH, gbuf0, sbuf0)
                        pltpu.async_copy(ytc_hbm.at[gbuf0], dbuf0, sem0)
                    pltpu.sync_copy(dbuf1, acc.at[sbuf1], add=True)
                return 0
            lax.fori_loop(0, (nch + 1) // 2, pair, 0)
            plsc.subcore_barrier()

            pltpu.sync_copy(acc.at[pl.ds(base, TPT)],
                            parts_hbm.at[c, t, pl.ds(h * HALF + base, TPT)])
            plsc.subcore_barrier()


@functools.cache
def _edge_kernel():
    return pl.kernel(
        _edge_body,
        out_type=jax.ShapeDtypeStruct((NC, R, 2 * HALF, D), jnp.float32),
        mesh=_sc_mesh(),
        compiler_params=pltpu.CompilerParams(needs_layout_passes=False),
        scratch_types=[
            pltpu.VMEM((CE,), jnp.int32),
            pltpu.VMEM((CE,), jnp.int32),
            pltpu.VMEM((CE,), jnp.int32),
            pltpu.VMEM((ACAP,), jnp.int32),
            pltpu.VMEM((ACAP,), jnp.int32),
            pltpu.VMEM((CH,), jnp.int32),
            pltpu.VMEM((CH,), jnp.int32),
            pltpu.VMEM((CH, D), jnp.float32),
            pltpu.VMEM((CH,), jnp.int32),
            pltpu.VMEM((CH,), jnp.int32),
            pltpu.VMEM((CH, D), jnp.float32),
            pltpu.VMEM_SHARED((HALF, D), jnp.float32),
            pltpu.SemaphoreType.DMA,
            pltpu.SemaphoreType.DMA,
        ],
    )


def _edge_call(row, col, typ, ytc):
    return _edge_kernel()(row, col, typ, ytc)


BN4 = 1000


def _comb_body(y_ref, parts_ref, degp_ref, o_ref):
    accv = y_ref[0]
    d2 = degp_ref[0] + degp_ref[1]
    dis = jnp.where(d2 > 0, lax.rsqrt(d2), 0.0)
    for t in range(R):
        accv = accv + (parts_ref[0, t] + parts_ref[1, t]) * dis[t]
    o_ref[...] = jnp.maximum(accv, 0.0)


def _comb_call(y, parts, degp):
    return pl.pallas_call(
        _comb_body,
        grid=(N // BN4,),
        in_specs=[
            pl.BlockSpec((1, BN4, D), lambda b: (0, b, 0)),
            pl.BlockSpec((NC, R, BN4, D), lambda b: (0, 0, b, 0)),
            pl.BlockSpec((NC, R, BN4, 1), lambda b: (0, 0, b, 0)),
        ],
        out_specs=pl.BlockSpec((BN4, D), lambda b: (b, 0)),
        out_shape=jax.ShapeDtypeStruct((N, D), jnp.float32),
    )(y, parts, degp)


def kernel(node_embs, mask, edge_index, edge_type, scorer_self, gates_self_W,
           gates_self_U, gates_self_b, scorer_rel, gates_rel_W, gates_rel_U,
           gates_rel_b, W_init_self, W_init_rel):
    x = node_embs
    row = edge_index[0].astype(jnp.int32)
    col = edge_index[1].astype(jnp.int32)
    typ = edge_type.astype(jnp.int32)

    p = jnp.concatenate(
        [scorer_self, jnp.moveaxis(scorer_rel, 0, 2).reshape(D, R)], axis=1)
    p = jnp.pad(p, ((0, 0), (0, HP - NH)))
    Wall = jnp.concatenate([gates_self_W[None], gates_rel_W], axis=0)
    Uall = jnp.concatenate([gates_self_U[None], gates_rel_U], axis=0)
    ball = jnp.concatenate([gates_self_b[None], gates_rel_b], axis=0)
    Qinit = jnp.concatenate([W_init_self[None], W_init_rel], axis=0)

    deg_raw = _deg_call(row, typ)
    wv, idx = _topk_call(x, p, mask.reshape(N, 1))
    Qall = _gru_call(idx, wv, x, Wall, Uall, ball, Qinit)
    degp = deg_raw[:, :DEG_BINS].reshape(NC, R, N, 1)
    y = _mm_call(x, Qall, degp)
    parts = _edge_call(row, col, typ, y.reshape(NH * N2, D))
    return _comb_call(y, parts, degp)


# speedup vs baseline: 23.1009x; 1.2215x over previous
import functools

import jax
import jax.numpy as jnp
from jax import lax
from jax.experimental import pallas as pl
from jax.experimental.pallas import tpu as pltpu
from jax.experimental.pallas import tpu_sc as plsc

N = 10000
E = 320000
D = 128
R = 4
K = 128
NH = R + 1
HP = 8

NC = 2
NS = 16
EP = E // (NC * NS)
HROWS = 79
DEG_BINS = R * N
DEG_PAD = 40960
CE = 2000
CH = 128
N2 = 10048
NL = 2 * R
HALF = 5120
TPT = HALF // NS
ACAP = EP + NL * CH + 48
ZROW = N

@functools.cache
def _sc_mesh():
    return plsc.VectorSubcoreMesh(core_axis_name="c", subcore_axis_name="s",
                                  num_cores=NC, num_subcores=NS)


def _deg_body(row_hbm, typ_hbm, deg_hbm, rbuf, tbuf, hlist, ones, zbuf, degacc):
    c = lax.axis_index("c")
    s = lax.axis_index("s")
    e0 = c * (E // NC) + s * EP

    for i in range(CH // 16):
        ones[pl.ds(i * 16, 16)] = jnp.ones((16,), jnp.float32)

    def zb(i, _):
        zbuf[pl.ds(i * 16, 16)] = jnp.zeros((16,), jnp.float32)
        return 0
    lax.fori_loop(0, (DEG_PAD // NS) // 16, zb, 0)

    pltpu.sync_copy(row_hbm.at[pl.ds(e0, EP)], rbuf)
    pltpu.sync_copy(typ_hbm.at[pl.ds(e0, EP)], tbuf)

    def hrow(j, _):
        for u in range(CH // 16):
            f = j * CH + u * 16
            hlist[j, pl.ds(u * 16, 16)] = (
                tbuf[pl.ds(f, 16)] * N + rbuf[pl.ds(f, 16)])
        return 0
    lax.fori_loop(0, HROWS - 1, hrow, 0)
    hlist[HROWS - 1, pl.ds(0, 16)] = (
        tbuf[pl.ds(EP - 16, 16)] * N + rbuf[pl.ds(EP - 16, 16)])
    for u in range(1, CH // 16):
        hlist[HROWS - 1, pl.ds(u * 16, 16)] = jnp.full((16,), DEG_BINS,
                                                       jnp.int32)

    pltpu.sync_copy(zbuf, degacc.at[pl.ds(s * (DEG_PAD // NS), DEG_PAD // NS)])
    plsc.subcore_barrier()

    def scat(j, _):
        pltpu.sync_copy(ones, degacc.at[hlist.at[j]], add=True)
        return 0
    lax.fori_loop(0, HROWS, scat, 0)
    plsc.subcore_barrier()

    sl = pl.ds(s * (DEG_PAD // NS), DEG_PAD // NS)
    pltpu.sync_copy(degacc.at[sl], deg_hbm.at[c, sl])


@functools.cache
def _deg_kernel():
    return pl.kernel(
        _deg_body,
        out_type=jax.ShapeDtypeStruct((NC, DEG_PAD), jnp.float32),
        mesh=_sc_mesh(),
        compiler_params=pltpu.CompilerParams(needs_layout_passes=False),
        scratch_types=[
            pltpu.VMEM((EP,), jnp.int32),
            pltpu.VMEM((EP,), jnp.int32),
            pltpu.VMEM((HROWS, CH), jnp.int32),
            pltpu.VMEM((CH,), jnp.float32),
            pltpu.VMEM((DEG_PAD // NS,), jnp.float32),
            pltpu.VMEM_SHARED((DEG_PAD,), jnp.float32),
        ],
    )


def _deg_call(row, typ):
    return _deg_kernel()(row, typ)


def _topk_body(x_ref, pt_ref, mask_ref, w_ref, idx_ref, s_ref):
    pt = pt_ref[...]
    nrm = jnp.sqrt(jnp.sum(pt * pt, axis=1, keepdims=True))
    sc = lax.dot_general(pt, x_ref[...], (((1,), (1,)), ((), ())),
                         preferred_element_type=jnp.float32)
    s_ref[...] = sc / (nrm + 1e-8) + mask_ref[...]
    iota = lax.broadcasted_iota(jnp.int32, (HP, N), 1)

    def body(i, _):
        sv = s_ref[...]
        m = jnp.max(sv, axis=1, keepdims=True)
        am = jnp.min(jnp.where(sv == m, iota, N), axis=1, keepdims=True)
        w_ref[pl.ds(i, 1), :] = jnp.tanh(m).reshape(1, HP)
        idx_ref[pl.ds(i, 1), :] = am.reshape(1, HP)
        s_ref[...] = jnp.where(iota == am, -jnp.inf, sv)
        return 0
    lax.fori_loop(0, K, body, 0)


def _topk_call(x, pt, mask2d):
    return pl.pallas_call(
        _topk_body,
        out_shape=[jax.ShapeDtypeStruct((K, HP), jnp.float32),
                   jax.ShapeDtypeStruct((K, HP), jnp.int32)],
        scratch_shapes=[pltpu.VMEM((HP, N), jnp.float32)],
    )(x, pt, mask2d)


def _gru_body(idx_ref, w_ref, x_ref, W_ref, U_ref, b_ref, q_ref, out_ref,
              sel_ref):
    h = pl.program_id(0)

    def gather(i, _):
        r = idx_ref[i, h]
        sel_ref[pl.ds(i, 1), :] = x_ref[pl.ds(r, 1), :] * w_ref[i, h]
        return 0
    lax.fori_loop(0, K, gather, 0)

    sw = sel_ref[...]
    q = q_ref[0]

    def nt(a, b):
        return lax.dot_general(a, b, (((1,), (1,)), ((), ())),
                               preferred_element_type=jnp.float32)

    def nn(a, b):
        return jnp.dot(a, b, preferred_element_type=jnp.float32)

    upd = jax.nn.sigmoid(nt(W_ref[0, 0], sw) + nn(U_ref[0, 0], q)
                         + b_ref[0, 0])
    rst = jax.nn.sigmoid(nt(W_ref[0, 1], sw) + nn(U_ref[0, 1], q)
                         + b_ref[0, 1])
    hc = jnp.tanh(nt(W_ref[0, 2], sw) + nn(U_ref[0, 2], rst * q)
                  + b_ref[0, 2])
    out_ref[0] = (1.0 - upd) * q + upd * hc


def _gru_call(idx, wv, x, Wall, Uall, ball, Qinit):
    return pl.pallas_call(
        _gru_body,
        grid=(NH,),
        in_specs=[
            pl.BlockSpec(memory_space=pltpu.SMEM),
            pl.BlockSpec(memory_space=pltpu.SMEM),
            pl.BlockSpec((N, D), lambda h: (0, 0)),
            pl.BlockSpec((1, 3, D, D), lambda h: (h, 0, 0, 0)),
            pl.BlockSpec((1, 3, D, D), lambda h: (h, 0, 0, 0)),
            pl.BlockSpec((1, 3, D, D), lambda h: (h, 0, 0, 0)),
            pl.BlockSpec((1, D, D), lambda h: (h, 0, 0)),
        ],
        out_specs=pl.BlockSpec((1, D, D), lambda h: (h, 0, 0)),
        out_shape=jax.ShapeDtypeStruct((NH, D, D), jnp.float32),
        scratch_shapes=[pltpu.VMEM((K, D), jnp.float32)],
    )(idx, wv, x, Wall, Uall, ball, Qinit)


BN = 2512


def _mm_body(x_ref, q_ref, degp_ref, y_ref):
    r = pl.program_id(0)
    b = pl.program_id(1)
    xq = jnp.dot(x_ref[...], q_ref[0], preferred_element_type=jnp.float32)
    d2 = degp_ref[0, 0] + degp_ref[1, 0]
    dis = jnp.where(d2 > 0, lax.rsqrt(d2), 0.0)
    scale = jnp.where(r == 0, jnp.ones_like(dis), dis)
    rid = b * BN + lax.broadcasted_iota(jnp.int32, (BN, 1), 0)
    y_ref[0] = jnp.where(rid < N, xq * scale, 0.0)


def _mm_call(x, Qall, degp):
    return pl.pallas_call(
        _mm_body,
        grid=(NH, N2 // BN),
        in_specs=[
            pl.BlockSpec((BN, D), lambda r, b: (b, 0)),
            pl.BlockSpec((1, D, D), lambda r, b: (r, 0, 0)),
            pl.BlockSpec((2, 1, BN, 1),
                         lambda r, b: (0, jnp.maximum(r - 1, 0), b, 0)),
        ],
        out_specs=pl.BlockSpec((1, BN, D), lambda r, b: (r, b, 0)),
        out_shape=jax.ShapeDtypeStruct((NH, N2, D), jnp.float32),
    )(x, Qall, degp)


def _edge_body(row_hbm, col_hbm, typ_hbm, ytc_hbm, parts_hbm,
               rbuf, cbuf, tbuf, garena, sarena, gbuf0, sbuf0, dbuf0,
               gbuf1, sbuf1, dbuf1, acc, sem0, sem1):
    c = lax.axis_index("c")
    s = lax.axis_index("s")
    e0 = c * (E // NC) + s * EP

    cnts = [jnp.int32(0)] * NL
    for ch in range(EP // CE):
        pltpu.sync_copy(row_hbm.at[pl.ds(e0 + ch * CE, CE)], rbuf)
        pltpu.sync_copy(typ_hbm.at[pl.ds(e0 + ch * CE, CE)], tbuf)

        def count_body(i, cn):
            rv = rbuf[pl.ds(i * 16, 16)]
            tv = tbuf[pl.ds(i * 16, 16)]
            hi = rv >= HALF
            new = []
            for t in range(R):
                for h in range(2):
                    m = (tv == t) & (hi if h else jnp.logical_not(hi))
                    new.append(cn[t * 2 + h] + jnp.max(
                        plsc.all_reduce_population_count(m)))
            return tuple(new)
        cnts = list(lax.fori_loop(0, CE // 16, count_body, tuple(cnts)))

    off = [jnp.int32(0)]
    for k in range(NL):
        off.append(off[k] + ((cnts[k] + CH - 1) & ~jnp.int32(CH - 1)))

    def pre(i, _):
        garena[pl.ds(i * 16, 16)] = jnp.full((16,), ZROW, jnp.int32)
        sarena[pl.ds(i * 16, 16)] = jnp.zeros((16,), jnp.int32)
        return 0
    lax.fori_loop(0, ACAP // 16, pre, 0)

    curs = list(off[:NL])
    for ch in range(EP // CE):
        pltpu.sync_copy(row_hbm.at[pl.ds(e0 + ch * CE, CE)], rbuf)
        pltpu.sync_copy(col_hbm.at[pl.ds(e0 + ch * CE, CE)], cbuf)
        pltpu.sync_copy(typ_hbm.at[pl.ds(e0 + ch * CE, CE)], tbuf)

        def fill_body(i, cu):
            rv = rbuf[pl.ds(i * 16, 16)]
            cv = cbuf[pl.ds(i * 16, 16)]
            tv = tbuf[pl.ds(i * 16, 16)]
            hi = rv >= HALF
            new = []
            for t in range(R):
                for h in range(2):
                    k = t * 2 + h
                    m = (tv == t) & (hi if h else jnp.logical_not(hi))
                    plsc.store_compressed(garena.at[pl.ds(cu[k], 16)],
                                          cv + (t + 1) * N2, mask=m)
                    plsc.store_compressed(sarena.at[pl.ds(cu[k], 16)],
                                          rv - h * HALF, mask=m)
                    new.append(cu[k] + jnp.max(
                        plsc.all_reduce_population_count(m)))
            return tuple(new)
        curs = list(lax.fori_loop(0, CE // 16, fill_body, tuple(curs)))

    base = s * TPT
    for h in range(2):
        for t in range(R):
            k = t * 2 + h

            def zdb(j, _):
                for u in range(D // 16):
                    dbuf0[j, pl.ds(u * 16, 16)] = jnp.zeros((16,), jnp.float32)
                return 0
            lax.fori_loop(0, CH, zdb, 0)
            for z in range(TPT // CH):
                pltpu.sync_copy(dbuf0, acc.at[pl.ds(base + z * CH, CH)])
            if TPT % CH:
                pltpu.sync_copy(dbuf0.at[pl.ds(0, TPT % CH)],
                                acc.at[pl.ds(base + (TPT // CH) * CH,
                                             TPT % CH)])
            plsc.subcore_barrier()

            nch = (off[k + 1] - off[k]) // CH

            def fill(o, gb, sb):
                for u in range(CH // 16):
                    gb[pl.ds(u * 16, 16)] = garena[pl.ds(o + u * 16, 16)]
                    sb[pl.ds(u * 16, 16)] = sarena[pl.ds(o + u * 16, 16)]

            @pl.when(nch > 0)
            def _():
                fill(off[k], gbuf0, sbuf0)
                pltpu.async_copy(ytc_hbm.at[gbuf0], dbuf0, sem0)

            def pair(m, _):
                j1 = 2 * m + 1
                pltpu.make_async_copy(ytc_hbm.at[gbuf0], dbuf0, sem0).wait()

                @pl.when(j1 < nch)
                def _():
                    fill(off[k] + j1 * CH, gbuf1, sbuf1)
                    pltpu.async_copy(ytc_hbm.at[gbuf1], dbuf1, sem1)
                pltpu.sync_copy(dbuf0, acc.at[sbuf0], add=True)

                @pl.when(j1 < nch)
                def _():
                    pltpu.make_async_copy(ytc_hbm.at[gbuf1], dbuf1,
                                          sem1).wait()

                    @pl.when(j1 + 1 < nch)
                    def _():
                        fill(off[k] + (j1 + 1) * CH, gbuf0, sbuf0)
                        pltpu.async_copy(ytc_hbm.at[gbuf0], dbuf0, sem0)
                    pltpu.sync_copy(dbuf1, acc.at[sbuf1], add=True)
                return 0
            lax.fori_loop(0, (nch + 1) // 2, pair, 0)
            plsc.subcore_barrier()

            pltpu.sync_copy(acc.at[pl.ds(base, TPT)],
                            parts_hbm.at[c, t, pl.ds(h * HALF + base, TPT)])
            plsc.subcore_barrier()


@functools.cache
def _edge_kernel():
    return pl.kernel(
        _edge_body,
        out_type=jax.ShapeDtypeStruct((NC, R, 2 * HALF, D), jnp.float32),
        mesh=_sc_mesh(),
        compiler_params=pltpu.CompilerParams(needs_layout_passes=False),
        scratch_types=[
            pltpu.VMEM((CE,), jnp.int32),
            pltpu.VMEM((CE,), jnp.int32),
            pltpu.VMEM((CE,), jnp.int32),
            pltpu.VMEM((ACAP,), jnp.int32),
            pltpu.VMEM((ACAP,), jnp.int32),
            pltpu.VMEM((CH,), jnp.int32),
            pltpu.VMEM((CH,), jnp.int32),
            pltpu.VMEM((CH, D), jnp.float32),
            pltpu.VMEM((CH,), jnp.int32),
            pltpu.VMEM((CH,), jnp.int32),
            pltpu.VMEM((CH, D), jnp.float32),
            pltpu.VMEM_SHARED((HALF, D), jnp.float32),
            pltpu.SemaphoreType.DMA,
            pltpu.SemaphoreType.DMA,
        ],
    )


def _edge_call(row, col, typ, ytc):
    return _edge_kernel()(row, col, typ, ytc)


BN4 = 1000


def _comb_body(y_ref, parts_ref, degp_ref, o_ref):
    accv = y_ref[0]
    d2 = degp_ref[0] + degp_ref[1]
    dis = jnp.where(d2 > 0, lax.rsqrt(d2), 0.0)
    for t in range(R):
        accv = accv + (parts_ref[0, t] + parts_ref[1, t]) * dis[t]
    o_ref[...] = jnp.maximum(accv, 0.0)


def _comb_call(y, parts, degp):
    return pl.pallas_call(
        _comb_body,
        grid=(N // BN4,),
        in_specs=[
            pl.BlockSpec((1, BN4, D), lambda b: (0, b, 0)),
            pl.BlockSpec((NC, R, BN4, D), lambda b: (0, 0, b, 0)),
            pl.BlockSpec((NC, R, BN4, 1), lambda b: (0, 0, b, 0)),
        ],
        out_specs=pl.BlockSpec((BN4, D), lambda b: (b, 0)),
        out_shape=jax.ShapeDtypeStruct((N, D), jnp.float32),
    )(y, parts, degp)


def kernel(node_embs, mask, edge_index, edge_type, scorer_self, gates_self_W,
           gates_self_U, gates_self_b, scorer_rel, gates_rel_W, gates_rel_U,
           gates_rel_b, W_init_self, W_init_rel):
    x = node_embs
    row = edge_index[0].astype(jnp.int32)
    col = edge_index[1].astype(jnp.int32)
    typ = edge_type.astype(jnp.int32)

    p = jnp.concatenate(
        [scorer_self, jnp.moveaxis(scorer_rel, 0, 2).reshape(D, R)], axis=1)
    p = jnp.pad(p, ((0, 0), (0, HP - NH)))
    Wall = jnp.concatenate([gates_self_W[None], gates_rel_W], axis=0)
    Uall = jnp.concatenate([gates_self_U[None], gates_rel_U], axis=0)
    ball = jnp.concatenate([gates_self_b[None], gates_rel_b], axis=0)
    Qinit = jnp.concatenate([W_init_self[None], W_init_rel], axis=0)

    deg_raw = _deg_call(row, typ)
    wv, idx = _topk_call(x, p.T, mask.reshape(1, N))
    Qall = _gru_call(idx, wv, x, Wall, Uall, ball, Qinit)
    degp = deg_raw[:, :DEG_BINS].reshape(NC, R, N, 1)
    y = _mm_call(x, Qall, degp)
    parts = _edge_call(row, col, typ, y.reshape(NH * N2, D))
    return _comb_call(y, parts, degp)
